# baseline (speedup 1.0000x reference)
import functools

import jax
import jax.numpy as jnp
from jax import lax
from jax.experimental import pallas as pl
from jax.experimental.pallas import tpu as pltpu
from jax.experimental.pallas import tpu_sc as plsc

VOCAB = 1000000
HIDDEN = 64
HPAD = 128
BATCH = 4096
SEQ = 200

NC = 2
NS = 16
NW = NC * NS

ROWS_PER_W = BATCH // NW
CHUNK = 2
N_CHUNKS = ROWS_PER_W // CHUNK
IDX_PER_CHUNK = CHUNK * SEQ
GSPLIT = 80
N_GATHERS = IDX_PER_CHUNK // GSPLIT
HREG = HIDDEN // 16
UNROLL = 4

TBN = 2048


def _fire_gathers(table_hbm, idx_buf, rows_buf, sem):
    return [
        pltpu.async_copy(
            table_hbm.at[idx_buf.at[pl.ds(i * GSPLIT, GSPLIT)]],
            rows_buf.at[pl.ds(i * GSPLIT, GSPLIT)],
            sem,
        )
        for i in range(N_GATHERS)
    ]


def _wait_gathers(table_hbm, idx_buf, rows_buf, sem):
    for i in range(N_GATHERS):
        pltpu.make_async_copy(
            table_hbm.at[idx_buf.at[pl.ds(i * GSPLIT, GSPLIT)]],
            rows_buf.at[pl.ds(i * GSPLIT, GSPLIT)],
            sem,
        ).wait()


def _accumulate(rows_buf, pooled_v, c):
    for r in range(CHUNK):
        def jbody(j, accs):
            accs = list(accs)
            for u in range(UNROLL):
                row = r * SEQ + j * UNROLL + u
                for k in range(HREG):
                    accs[k] = accs[k] + rows_buf[row, pl.ds(k * 16, 16)]
            return tuple(accs)

        accs = lax.fori_loop(
            0, SEQ // UNROLL, jbody,
            tuple(jnp.zeros((16,), jnp.float32) for _ in range(HREG)),
        )
        out_base = (c * CHUNK + r) * HIDDEN
        for k in range(HREG):
            pooled_v[pl.ds(out_base + k * 16, 16)] = accs[k]


def _sc_pool_kernel(xflat_hbm, table_hbm, out_hbm,
                    idx_a, idx_b, rows_a, rows_b, pooled_v, semg, semi):
    wid = lax.axis_index("s") * NC + lax.axis_index("c")
    base_flat = wid * (ROWS_PER_W * SEQ)

    def idx_src(c):
        return xflat_hbm.at[pl.ds(base_flat + c * IDX_PER_CHUNK, IDX_PER_CHUNK)]

    def clamp(c):
        return jnp.minimum(c, N_CHUNKS - 1)

    pltpu.sync_copy(idx_src(0), idx_a)
    _fire_gathers(table_hbm, idx_a, rows_a, semg)
    pltpu.async_copy(idx_src(1), idx_b, semi)

    def body(i, carry):
        c0 = 2 * i
        pltpu.make_async_copy(idx_src(clamp(c0 + 1)), idx_b, semi).wait()
        _wait_gathers(table_hbm, idx_a, rows_a, semg)
        _fire_gathers(table_hbm, idx_b, rows_b, semg)
        pltpu.async_copy(idx_src(clamp(c0 + 2)), idx_a, semi)
        _accumulate(rows_a, pooled_v, c0)
        pltpu.make_async_copy(idx_src(clamp(c0 + 2)), idx_a, semi).wait()
        _wait_gathers(table_hbm, idx_b, rows_b, semg)
        _fire_gathers(table_hbm, idx_a, rows_a, semg)
        pltpu.async_copy(idx_src(clamp(c0 + 3)), idx_b, semi)
        _accumulate(rows_b, pooled_v, c0 + 1)
        return carry

    lax.fori_loop(0, N_CHUNKS // 2, body, 0)
    _wait_gathers(table_hbm, idx_a, rows_a, semg)
    pltpu.make_async_copy(idx_src(N_CHUNKS - 1), idx_b, semi).wait()
    pltpu.sync_copy(
        pooled_v, out_hbm.at[pl.ds(wid * (ROWS_PER_W * HIDDEN), ROWS_PER_W * HIDDEN)]
    )


def _sc_pool(xflat, table):
    mesh = plsc.VectorSubcoreMesh(core_axis_name="c", subcore_axis_name="s")
    k = functools.partial(
        pl.kernel,
        mesh=mesh,
        out_type=jax.ShapeDtypeStruct((BATCH * HIDDEN,), jnp.float32),
        scratch_types=[
            pltpu.VMEM((IDX_PER_CHUNK,), jnp.int32),
            pltpu.VMEM((IDX_PER_CHUNK,), jnp.int32),
            pltpu.VMEM((IDX_PER_CHUNK, HPAD), jnp.float32),
            pltpu.VMEM((IDX_PER_CHUNK, HPAD), jnp.float32),
            pltpu.VMEM((ROWS_PER_W * HIDDEN,), jnp.float32),
            pltpu.SemaphoreType.DMA,
            pltpu.SemaphoreType.DMA,
        ],
    )(_sc_pool_kernel)
    return k(xflat, table)


def _tpad_kernel(tin_ref, eye_ref, out_ref):
    t = lax.dot_general(
        tin_ref[...], eye_ref[...], (((0,), (0,)), ((), ())),
        precision=lax.Precision.HIGHEST,
        preferred_element_type=jnp.float32,
    )
    out_ref[...] = jnp.concatenate([t, t], axis=1)


def _transpose_pad(table_t):
    eye = jnp.eye(HIDDEN, dtype=jnp.float32)
    return pl.pallas_call(
        _tpad_kernel,
        grid=(pl.cdiv(VOCAB, TBN),),
        in_specs=[
            pl.BlockSpec((HIDDEN, TBN), lambda i: (0, i)),
            pl.BlockSpec((HIDDEN, HIDDEN), lambda i: (0, 0)),
        ],
        out_specs=pl.BlockSpec((TBN, HPAD), lambda i: (i, 0)),
        out_shape=jax.ShapeDtypeStruct((VOCAB, HPAD), jnp.float32),
    )(table_t, eye)


def _fc_kernel(p_ref, wt_ref, b_ref, o_ref):
    p = p_ref[...] * (1.0 / SEQ)
    acc = jnp.dot(p, wt_ref[...], preferred_element_type=jnp.float32)
    o_ref[...] = jnp.maximum(acc + b_ref[...], 0.0)


def kernel(x, emb_table, W, b):
    xflat = x.astype(jnp.int32).reshape(-1)
    table128 = _transpose_pad(emb_table.T)
    pooled = _sc_pool(xflat, table128).reshape(BATCH, HIDDEN)
    out = pl.pallas_call(
        _fc_kernel,
        out_shape=jax.ShapeDtypeStruct((BATCH, HIDDEN), jnp.float32),
    )(pooled, W.T, b.reshape(1, HIDDEN))
    return out


# speedup vs baseline: 1.5371x; 1.5371x over previous
import functools

import jax
import jax.numpy as jnp
from jax import lax
from jax.experimental import pallas as pl
from jax.experimental.pallas import tpu as pltpu
from jax.experimental.pallas import tpu_sc as plsc

VOCAB = 1000000
HIDDEN = 64
HPAD = 128
BATCH = 4096
SEQ = 200

NC = 2
NS = 16
NW = NC * NS

ROWS_PER_W = BATCH // NW
CHUNK = 2
N_CHUNKS = ROWS_PER_W // CHUNK
IDX_PER_CHUNK = CHUNK * SEQ
IDX_BUF = IDX_PER_CHUNK + 16
GSPLIT = 80
N_GATHERS = IDX_PER_CHUNK // GSPLIT
HREG = HIDDEN // 16
UNROLL = 8

TBN = 2048
NBLK_MAIN = 244
OFF = TBN * NBLK_MAIN
TAIL = OFF * 2
N_LINES = TBN * (NBLK_MAIN + 1)


def _fire_gathers(table_hbm, idx_buf, rows_buf, sem):
    return [
        pltpu.async_copy(
            table_hbm.at[idx_buf.at[pl.ds(i * GSPLIT, GSPLIT)]],
            rows_buf.at[pl.ds(i * GSPLIT, GSPLIT)],
            sem,
        )
        for i in range(N_GATHERS)
    ]


def _wait_gathers(table_hbm, idx_buf, rows_buf, sem):
    for i in range(N_GATHERS):
        pltpu.make_async_copy(
            table_hbm.at[idx_buf.at[pl.ds(i * GSPLIT, GSPLIT)]],
            rows_buf.at[pl.ds(i * GSPLIT, GSPLIT)],
            sem,
        ).wait()


def _accumulate(rows_buf, par_buf, pooled_v, c):
    for r in range(CHUNK):
        def jbody(j, accs):
            accs = list(accs)
            base = r * SEQ + j * UNROLL
            pvec = par_buf[pl.ds(base, 16)]
            for u in range(UNROLL):
                row = base + u
                off = (pvec[u] & 1) * HIDDEN
                for k in range(HREG):
                    accs[k] = accs[k] + rows_buf[row, pl.ds(off + k * 16, 16)]
            return tuple(accs)

        accs = lax.fori_loop(
            0, SEQ // UNROLL, jbody,
            tuple(jnp.zeros((16,), jnp.float32) for _ in range(HREG)),
        )
        out_base = (c * CHUNK + r) * HIDDEN
        for k in range(HREG):
            pooled_v[pl.ds(out_base + k * 16, 16)] = accs[k]


def _copy_par(par_buf, par_hold):
    for i in range(IDX_BUF // 16):
        par_hold[pl.ds(i * 16, 16)] = par_buf[pl.ds(i * 16, 16)]


def _sc_pool_kernel(xh_hbm, xp_hbm, table_hbm, out_hbm,
                    idx_a, idx_b, par_a, par_b, par_hold, rows_a, rows_b,
                    pooled_v, semg, semi):
    wid = lax.axis_index("s") * NC + lax.axis_index("c")
    base_flat = wid * (ROWS_PER_W * SEQ)

    def idx_src(c):
        return xh_hbm.at[pl.ds(base_flat + c * IDX_PER_CHUNK, IDX_PER_CHUNK)]

    def par_src(c):
        return xp_hbm.at[pl.ds(base_flat + c * IDX_PER_CHUNK, IDX_PER_CHUNK)]

    def clamp(c):
        return jnp.minimum(c, N_CHUNKS - 1)

    def fire_idx(c, idx_buf, par_buf):
        pltpu.async_copy(idx_src(c), idx_buf.at[pl.ds(0, IDX_PER_CHUNK)], semi)
        pltpu.async_copy(par_src(c), par_buf.at[pl.ds(0, IDX_PER_CHUNK)], semi)

    def wait_idx(c, idx_buf, par_buf):
        pltpu.make_async_copy(
            idx_src(c), idx_buf.at[pl.ds(0, IDX_PER_CHUNK)], semi).wait()
        pltpu.make_async_copy(
            par_src(c), par_buf.at[pl.ds(0, IDX_PER_CHUNK)], semi).wait()

    pltpu.sync_copy(idx_src(0), idx_a.at[pl.ds(0, IDX_PER_CHUNK)])
    pltpu.sync_copy(par_src(0), par_a.at[pl.ds(0, IDX_PER_CHUNK)])
    _fire_gathers(table_hbm, idx_a, rows_a, semg)
    fire_idx(1, idx_b, par_b)

    def body(i, carry):
        c0 = 2 * i
        wait_idx(clamp(c0 + 1), idx_b, par_b)
        _wait_gathers(table_hbm, idx_a, rows_a, semg)
        _fire_gathers(table_hbm, idx_b, rows_b, semg)
        _copy_par(par_a, par_hold)
        fire_idx(clamp(c0 + 2), idx_a, par_a)
        _accumulate(rows_a, par_hold, pooled_v, c0)
        wait_idx(clamp(c0 + 2), idx_a, par_a)
        _wait_gathers(table_hbm, idx_b, rows_b, semg)
        _fire_gathers(table_hbm, idx_a, rows_a, semg)
        _copy_par(par_b, par_hold)
        fire_idx(clamp(c0 + 3), idx_b, par_b)
        _accumulate(rows_b, par_hold, pooled_v, c0 + 1)
        return carry

    lax.fori_loop(0, N_CHUNKS // 2, body, 0)
    _wait_gathers(table_hbm, idx_a, rows_a, semg)
    wait_idx(N_CHUNKS - 1, idx_b, par_b)
    pltpu.sync_copy(
        pooled_v, out_hbm.at[pl.ds(wid * (ROWS_PER_W * HIDDEN), ROWS_PER_W * HIDDEN)]
    )


def _sc_pool(xh, xp, table):
    mesh = plsc.VectorSubcoreMesh(core_axis_name="c", subcore_axis_name="s")
    k = functools.partial(
        pl.kernel,
        mesh=mesh,
        out_type=jax.ShapeDtypeStruct((BATCH * HIDDEN,), jnp.float32),
        scratch_types=[
            pltpu.VMEM((IDX_BUF,), jnp.int32),
            pltpu.VMEM((IDX_BUF,), jnp.int32),
            pltpu.VMEM((IDX_BUF,), jnp.int32),
            pltpu.VMEM((IDX_BUF,), jnp.int32),
            pltpu.VMEM((IDX_BUF,), jnp.int32),
            pltpu.VMEM((IDX_PER_CHUNK, HPAD), jnp.float32),
            pltpu.VMEM((IDX_PER_CHUNK, HPAD), jnp.float32),
            pltpu.VMEM((ROWS_PER_W * HIDDEN,), jnp.float32),
            pltpu.SemaphoreType.DMA,
            pltpu.SemaphoreType.DMA,
        ],
    )(_sc_pool_kernel)
    return k(xh, xp, table)


def _tpad_kernel(tlo_ref, thi_ref, eye_ref, out_ref):
    dn = (((0,), (0,)), ((), ()))
    tlo = lax.dot_general(tlo_ref[...], eye_ref[...], dn,
                          preferred_element_type=jnp.float32)
    thi = lax.dot_general(thi_ref[...], eye_ref[...], dn,
                          preferred_element_type=jnp.float32)
    out_ref[...] = jnp.concatenate([tlo, thi], axis=1)


def _transpose_pair(table_t):
    eye = jnp.eye(HIDDEN, dtype=jnp.float32)
    return pl.pallas_call(
        _tpad_kernel,
        grid=(NBLK_MAIN + 1,),
        in_specs=[
            pl.BlockSpec(
                (HIDDEN, TBN),
                lambda i: (0, jnp.where(i < NBLK_MAIN, i, 2 * NBLK_MAIN)),
            ),
            pl.BlockSpec(
                (HIDDEN, TBN),
                lambda i: (0, jnp.where(i < NBLK_MAIN, i + NBLK_MAIN, 2 * NBLK_MAIN)),
            ),
            pl.BlockSpec((HIDDEN, HIDDEN), lambda i: (0, 0)),
        ],
        out_specs=pl.BlockSpec((TBN, HPAD), lambda i: (i, 0)),
        out_shape=jax.ShapeDtypeStruct((N_LINES, HPAD), jnp.float32),
    )(table_t, table_t, eye)


def _fc_kernel(p_ref, wt_ref, b_ref, o_ref):
    p = p_ref[...] * (1.0 / SEQ)
    acc = jnp.dot(p, wt_ref[...], preferred_element_type=jnp.float32)
    o_ref[...] = jnp.maximum(acc + b_ref[...], 0.0)


def kernel(x, emb_table, W, b):
    xflat = x.astype(jnp.int32).reshape(-1)
    table_pairs = _transpose_pair(emb_table.T)
    is_tail = xflat >= TAIL
    xp = ((xflat >= OFF) & ~is_tail).astype(jnp.int32)
    xh = jnp.where(is_tail, xflat - TAIL + OFF, xflat - OFF * xp)
    pooled = _sc_pool(xh, xp, table_pairs).reshape(BATCH, HIDDEN)
    out = pl.pallas_call(
        _fc_kernel,
        out_shape=jax.ShapeDtypeStruct((BATCH, HIDDEN), jnp.float32),
    )(pooled, W.T, b.reshape(1, HIDDEN))
    return out


# speedup vs baseline: 1.7523x; 1.1400x over previous
import functools

import jax
import jax.numpy as jnp
from jax import lax
from jax.experimental import pallas as pl
from jax.experimental.pallas import tpu as pltpu
from jax.experimental.pallas import tpu_sc as plsc

VOCAB = 1000000
HIDDEN = 64
HPAD = 128
BATCH = 4096
SEQ = 200

NC = 2
NS = 16
NW = NC * NS

ROWS_PER_W = BATCH // NW
CHUNK = 2
N_CHUNKS = ROWS_PER_W // CHUNK
IDX_PER_CHUNK = CHUNK * SEQ
IDX_BUF = IDX_PER_CHUNK + 16
GSPLIT = 80
N_GATHERS = IDX_PER_CHUNK // GSPLIT
HREG = HIDDEN // 16
UNROLL = 8

TBN = 4096
NBLK_MAIN = 122
OFF = TBN * NBLK_MAIN
TAIL = OFF * 2
N_LINES = TBN * (NBLK_MAIN + 1)


def _fire_gathers(table_hbm, idx_buf, rows_buf, sem):
    return [
        pltpu.async_copy(
            table_hbm.at[idx_buf.at[pl.ds(i * GSPLIT, GSPLIT)]],
            rows_buf.at[pl.ds(i * GSPLIT, GSPLIT)],
            sem,
        )
        for i in range(N_GATHERS)
    ]


def _wait_gathers(table_hbm, idx_buf, rows_buf, sem):
    for i in range(N_GATHERS):
        pltpu.make_async_copy(
            table_hbm.at[idx_buf.at[pl.ds(i * GSPLIT, GSPLIT)]],
            rows_buf.at[pl.ds(i * GSPLIT, GSPLIT)],
            sem,
        ).wait()


def _accumulate(rows_buf, par_buf, pooled_v, c):
    for r in range(CHUNK):
        def jbody(j, accs):
            accs = list(accs)
            base = r * SEQ + j * UNROLL
            pvec = par_buf[pl.ds(base, 16)]
            for u in range(UNROLL):
                row = base + u
                off = (pvec[u] & 1) * HIDDEN
                for k in range(HREG):
                    accs[k] = accs[k] + rows_buf[row, pl.ds(off + k * 16, 16)]
            return tuple(accs)

        accs = lax.fori_loop(
            0, SEQ // UNROLL, jbody,
            tuple(jnp.zeros((16,), jnp.float32) for _ in range(HREG)),
        )
        out_base = (c * CHUNK + r) * HIDDEN
        for k in range(HREG):
            pooled_v[pl.ds(out_base + k * 16, 16)] = accs[k]


def _copy_par(par_buf, par_hold):
    for i in range(IDX_BUF // 16):
        par_hold[pl.ds(i * 16, 16)] = par_buf[pl.ds(i * 16, 16)]


def _sc_pool_kernel(xh_hbm, xp_hbm, table_hbm, out_hbm,
                    idx_a, idx_b, par_a, par_b, par_hold, rows_a, rows_b,
                    pooled_v, semg, semi):
    wid = lax.axis_index("s") * NC + lax.axis_index("c")
    base_flat = wid * (ROWS_PER_W * SEQ)

    def idx_src(c):
        return xh_hbm.at[pl.ds(base_flat + c * IDX_PER_CHUNK, IDX_PER_CHUNK)]

    def par_src(c):
        return xp_hbm.at[pl.ds(base_flat + c * IDX_PER_CHUNK, IDX_PER_CHUNK)]

    def clamp(c):
        return jnp.minimum(c, N_CHUNKS - 1)

    def fire_idx(c, idx_buf, par_buf):
        pltpu.async_copy(idx_src(c), idx_buf.at[pl.ds(0, IDX_PER_CHUNK)], semi)
        pltpu.async_copy(par_src(c), par_buf.at[pl.ds(0, IDX_PER_CHUNK)], semi)

    def wait_idx(c, idx_buf, par_buf):
        pltpu.make_async_copy(
            idx_src(c), idx_buf.at[pl.ds(0, IDX_PER_CHUNK)], semi).wait()
        pltpu.make_async_copy(
            par_src(c), par_buf.at[pl.ds(0, IDX_PER_CHUNK)], semi).wait()

    pltpu.sync_copy(idx_src(0), idx_a.at[pl.ds(0, IDX_PER_CHUNK)])
    pltpu.sync_copy(par_src(0), par_a.at[pl.ds(0, IDX_PER_CHUNK)])
    _fire_gathers(table_hbm, idx_a, rows_a, semg)
    fire_idx(1, idx_b, par_b)

    def body(i, carry):
        c0 = 2 * i
        wait_idx(clamp(c0 + 1), idx_b, par_b)
        _wait_gathers(table_hbm, idx_a, rows_a, semg)
        _fire_gathers(table_hbm, idx_b, rows_b, semg)
        _copy_par(par_a, par_hold)
        fire_idx(clamp(c0 + 2), idx_a, par_a)
        _accumulate(rows_a, par_hold, pooled_v, c0)
        wait_idx(clamp(c0 + 2), idx_a, par_a)
        _wait_gathers(table_hbm, idx_b, rows_b, semg)
        _fire_gathers(table_hbm, idx_a, rows_a, semg)
        _copy_par(par_b, par_hold)
        fire_idx(clamp(c0 + 3), idx_b, par_b)
        _accumulate(rows_b, par_hold, pooled_v, c0 + 1)
        return carry

    lax.fori_loop(0, N_CHUNKS // 2, body, 0)
    _wait_gathers(table_hbm, idx_a, rows_a, semg)
    wait_idx(N_CHUNKS - 1, idx_b, par_b)
    pltpu.sync_copy(
        pooled_v, out_hbm.at[pl.ds(wid * (ROWS_PER_W * HIDDEN), ROWS_PER_W * HIDDEN)]
    )


def _sc_pool(xh, xp, table):
    mesh = plsc.VectorSubcoreMesh(core_axis_name="c", subcore_axis_name="s")
    k = functools.partial(
        pl.kernel,
        mesh=mesh,
        out_type=jax.ShapeDtypeStruct((BATCH * HIDDEN,), jnp.float32),
        scratch_types=[
            pltpu.VMEM((IDX_BUF,), jnp.int32),
            pltpu.VMEM((IDX_BUF,), jnp.int32),
            pltpu.VMEM((IDX_BUF,), jnp.int32),
            pltpu.VMEM((IDX_BUF,), jnp.int32),
            pltpu.VMEM((IDX_BUF,), jnp.int32),
            pltpu.VMEM((IDX_PER_CHUNK, HPAD), jnp.float32),
            pltpu.VMEM((IDX_PER_CHUNK, HPAD), jnp.float32),
            pltpu.VMEM((ROWS_PER_W * HIDDEN,), jnp.float32),
            pltpu.SemaphoreType.DMA,
            pltpu.SemaphoreType.DMA,
        ],
    )(_sc_pool_kernel)
    return k(xh, xp, table)


def _tpad_kernel(tlo_ref, thi_ref, eye_ref, out_ref):
    dn = (((0,), (0,)), ((), ()))
    tlo = lax.dot_general(tlo_ref[...], eye_ref[...], dn,
                          preferred_element_type=jnp.float32)
    thi = lax.dot_general(thi_ref[...], eye_ref[...], dn,
                          preferred_element_type=jnp.float32)
    out_ref[...] = jnp.concatenate([tlo, thi], axis=1)


def _transpose_pair(table_t):
    eye = jnp.eye(HIDDEN, dtype=jnp.float32)
    return pl.pallas_call(
        _tpad_kernel,
        grid=(NBLK_MAIN + 1,),
        in_specs=[
            pl.BlockSpec(
                (HIDDEN, TBN),
                lambda i: (0, jnp.where(i < NBLK_MAIN, i, 2 * NBLK_MAIN)),
            ),
            pl.BlockSpec(
                (HIDDEN, TBN),
                lambda i: (0, jnp.where(i < NBLK_MAIN, i + NBLK_MAIN, 2 * NBLK_MAIN)),
            ),
            pl.BlockSpec((HIDDEN, HIDDEN), lambda i: (0, 0)),
        ],
        out_specs=pl.BlockSpec((TBN, HPAD), lambda i: (i, 0)),
        out_shape=jax.ShapeDtypeStruct((N_LINES, HPAD), jnp.float32),
    )(table_t, table_t, eye)


def _fc_kernel(p_ref, wt_ref, b_ref, o_ref):
    p = p_ref[...] * (1.0 / SEQ)
    acc = jnp.dot(p, wt_ref[...], preferred_element_type=jnp.float32)
    o_ref[...] = jnp.maximum(acc + b_ref[...], 0.0)


def kernel(x, emb_table, W, b):
    xflat = x.astype(jnp.int32).reshape(-1)
    table_pairs = _transpose_pair(emb_table.T)
    is_tail = xflat >= TAIL
    xp = ((xflat >= OFF) & ~is_tail).astype(jnp.int32)
    xh = jnp.where(is_tail, xflat - TAIL + OFF, xflat - OFF * xp)
    pooled = _sc_pool(xh, xp, table_pairs).reshape(BATCH, HIDDEN)
    out = pl.pallas_call(
        _fc_kernel,
        out_shape=jax.ShapeDtypeStruct((BATCH, HIDDEN), jnp.float32),
    )(pooled, W.T, b.reshape(1, HIDDEN))
    return out


# speedup vs baseline: 1.8765x; 1.0709x over previous
import functools

import jax
import jax.numpy as jnp
from jax import lax
from jax.experimental import pallas as pl
from jax.experimental.pallas import tpu as pltpu
from jax.experimental.pallas import tpu_sc as plsc

VOCAB = 1000000
HIDDEN = 64
HPAD = 128
BATCH = 4096
SEQ = 200

NC = 2
NS = 16
NW = NC * NS

ROWS_PER_W = BATCH // NW
CHUNK = 2
N_CHUNKS = ROWS_PER_W // CHUNK
IDX_PER_CHUNK = CHUNK * SEQ
IDX_BUF = IDX_PER_CHUNK + 16
GSPLIT = 80
N_GATHERS = IDX_PER_CHUNK // GSPLIT
HREG = HIDDEN // 16
UNROLL = 8

TBN = 8192
NBLK_MAIN = 61
OFF = TBN * NBLK_MAIN
TAIL = OFF * 2
N_LINES = TBN * (NBLK_MAIN + 1)


def _fire_gathers(table_hbm, idx_buf, rows_buf, sem):
    return [
        pltpu.async_copy(
            table_hbm.at[idx_buf.at[pl.ds(i * GSPLIT, GSPLIT)]],
            rows_buf.at[pl.ds(i * GSPLIT, GSPLIT)],
            sem,
        )
        for i in range(N_GATHERS)
    ]


def _wait_gathers(table_hbm, idx_buf, rows_buf, sem):
    for i in range(N_GATHERS):
        pltpu.make_async_copy(
            table_hbm.at[idx_buf.at[pl.ds(i * GSPLIT, GSPLIT)]],
            rows_buf.at[pl.ds(i * GSPLIT, GSPLIT)],
            sem,
        ).wait()


def _accumulate(rows_buf, par_buf, pooled_v, c):
    for r in range(CHUNK):
        def jbody(j, accs):
            accs = list(accs)
            base = r * SEQ + j * UNROLL
            pvec = par_buf[pl.ds(base, 16)]
            for u in range(UNROLL):
                row = base + u
                off = (pvec[u] & 1) * HIDDEN
                for k in range(HREG):
                    accs[k] = accs[k] + rows_buf[row, pl.ds(off + k * 16, 16)]
            return tuple(accs)

        accs = lax.fori_loop(
            0, SEQ // UNROLL, jbody,
            tuple(jnp.zeros((16,), jnp.float32) for _ in range(HREG)),
        )
        out_base = (c * CHUNK + r) * HIDDEN
        for k in range(HREG):
            pooled_v[pl.ds(out_base + k * 16, 16)] = accs[k]


def _copy_par(par_buf, par_hold):
    for i in range(IDX_BUF // 16):
        par_hold[pl.ds(i * 16, 16)] = par_buf[pl.ds(i * 16, 16)]


def _sc_pool_kernel(xh_hbm, xp_hbm, table_hbm, out_hbm,
                    idx_a, idx_b, par_a, par_b, par_hold, rows_a, rows_b,
                    pooled_v, semg, semi):
    wid = lax.axis_index("s") * NC + lax.axis_index("c")
    base_flat = wid * (ROWS_PER_W * SEQ)

    def idx_src(c):
        return xh_hbm.at[pl.ds(base_flat + c * IDX_PER_CHUNK, IDX_PER_CHUNK)]

    def par_src(c):
        return xp_hbm.at[pl.ds(base_flat + c * IDX_PER_CHUNK, IDX_PER_CHUNK)]

    def clamp(c):
        return jnp.minimum(c, N_CHUNKS - 1)

    def fire_idx(c, idx_buf, par_buf):
        pltpu.async_copy(idx_src(c), idx_buf.at[pl.ds(0, IDX_PER_CHUNK)], semi)
        pltpu.async_copy(par_src(c), par_buf.at[pl.ds(0, IDX_PER_CHUNK)], semi)

    def wait_idx(c, idx_buf, par_buf):
        pltpu.make_async_copy(
            idx_src(c), idx_buf.at[pl.ds(0, IDX_PER_CHUNK)], semi).wait()
        pltpu.make_async_copy(
            par_src(c), par_buf.at[pl.ds(0, IDX_PER_CHUNK)], semi).wait()

    pltpu.sync_copy(idx_src(0), idx_a.at[pl.ds(0, IDX_PER_CHUNK)])
    pltpu.sync_copy(par_src(0), par_a.at[pl.ds(0, IDX_PER_CHUNK)])
    _fire_gathers(table_hbm, idx_a, rows_a, semg)
    fire_idx(1, idx_b, par_b)

    def body(i, carry):
        c0 = 2 * i
        wait_idx(clamp(c0 + 1), idx_b, par_b)
        _wait_gathers(table_hbm, idx_a, rows_a, semg)
        _fire_gathers(table_hbm, idx_b, rows_b, semg)
        _copy_par(par_a, par_hold)
        fire_idx(clamp(c0 + 2), idx_a, par_a)
        _accumulate(rows_a, par_hold, pooled_v, c0)
        wait_idx(clamp(c0 + 2), idx_a, par_a)
        _wait_gathers(table_hbm, idx_b, rows_b, semg)
        _fire_gathers(table_hbm, idx_a, rows_a, semg)
        _copy_par(par_b, par_hold)
        fire_idx(clamp(c0 + 3), idx_b, par_b)
        _accumulate(rows_b, par_hold, pooled_v, c0 + 1)
        return carry

    lax.fori_loop(0, N_CHUNKS // 2, body, 0)
    _wait_gathers(table_hbm, idx_a, rows_a, semg)
    wait_idx(N_CHUNKS - 1, idx_b, par_b)
    pltpu.sync_copy(
        pooled_v, out_hbm.at[pl.ds(wid * (ROWS_PER_W * HIDDEN), ROWS_PER_W * HIDDEN)]
    )


def _sc_pool(xh, xp, table):
    mesh = plsc.VectorSubcoreMesh(core_axis_name="c", subcore_axis_name="s")
    k = functools.partial(
        pl.kernel,
        mesh=mesh,
        out_type=jax.ShapeDtypeStruct((BATCH * HIDDEN,), jnp.float32),
        scratch_types=[
            pltpu.VMEM((IDX_BUF,), jnp.int32),
            pltpu.VMEM((IDX_BUF,), jnp.int32),
            pltpu.VMEM((IDX_BUF,), jnp.int32),
            pltpu.VMEM((IDX_BUF,), jnp.int32),
            pltpu.VMEM((IDX_BUF,), jnp.int32),
            pltpu.VMEM((IDX_PER_CHUNK, HPAD), jnp.float32),
            pltpu.VMEM((IDX_PER_CHUNK, HPAD), jnp.float32),
            pltpu.VMEM((ROWS_PER_W * HIDDEN,), jnp.float32),
            pltpu.SemaphoreType.DMA,
            pltpu.SemaphoreType.DMA,
        ],
    )(_sc_pool_kernel)
    return k(xh, xp, table)


def _tpad_kernel(tlo_ref, thi_ref, eye_ref, out_ref):
    dn = (((0,), (0,)), ((), ()))
    tlo = lax.dot_general(tlo_ref[...], eye_ref[...], dn,
                          preferred_element_type=jnp.float32)
    thi = lax.dot_general(thi_ref[...], eye_ref[...], dn,
                          preferred_element_type=jnp.float32)
    out_ref[...] = jnp.concatenate([tlo, thi], axis=1)


def _transpose_pair(table_t):
    eye = jnp.eye(HIDDEN, dtype=jnp.float32)
    return pl.pallas_call(
        _tpad_kernel,
        grid=(NBLK_MAIN + 1,),
        in_specs=[
            pl.BlockSpec(
                (HIDDEN, TBN),
                lambda i: (0, jnp.where(i < NBLK_MAIN, i, 2 * NBLK_MAIN)),
            ),
            pl.BlockSpec(
                (HIDDEN, TBN),
                lambda i: (0, jnp.where(i < NBLK_MAIN, i + NBLK_MAIN, 2 * NBLK_MAIN)),
            ),
            pl.BlockSpec((HIDDEN, HIDDEN), lambda i: (0, 0)),
        ],
        out_specs=pl.BlockSpec((TBN, HPAD), lambda i: (i, 0)),
        out_shape=jax.ShapeDtypeStruct((N_LINES, HPAD), jnp.float32),
    )(table_t, table_t, eye)


def _fc_kernel(p_ref, wt_ref, b_ref, o_ref):
    p = p_ref[...] * (1.0 / SEQ)
    acc = jnp.dot(p, wt_ref[...], preferred_element_type=jnp.float32)
    o_ref[...] = jnp.maximum(acc + b_ref[...], 0.0)


def kernel(x, emb_table, W, b):
    xflat = x.astype(jnp.int32).reshape(-1)
    table_pairs = _transpose_pair(emb_table.T)
    is_tail = xflat >= TAIL
    xp = ((xflat >= OFF) & ~is_tail).astype(jnp.int32)
    xh = jnp.where(is_tail, xflat - TAIL + OFF, xflat - OFF * xp)
    pooled = _sc_pool(xh, xp, table_pairs).reshape(BATCH, HIDDEN)
    out = pl.pallas_call(
        _fc_kernel,
        out_shape=jax.ShapeDtypeStruct((BATCH, HIDDEN), jnp.float32),
    )(pooled, W.T, b.reshape(1, HIDDEN))
    return out


# speedup vs baseline: 1.9106x; 1.0182x over previous
import functools

import jax
import jax.numpy as jnp
from jax import lax
from jax.experimental import pallas as pl
from jax.experimental.pallas import tpu as pltpu
from jax.experimental.pallas import tpu_sc as plsc

VOCAB = 1000000
HIDDEN = 64
HPAD = 128
BATCH = 4096
SEQ = 200

NC = 2
NS = 16
NW = NC * NS

ROWS_PER_W = BATCH // NW
CHUNK = 2
N_CHUNKS = ROWS_PER_W // CHUNK
IDX_PER_CHUNK = CHUNK * SEQ
IDX_BUF = IDX_PER_CHUNK + 16
GSPLIT = 80
N_GATHERS = IDX_PER_CHUNK // GSPLIT
HREG = HIDDEN // 16
UNROLL = 8

TBN = 16384
NBLK_MAIN = 30
N_TAIL_BLK = 2
OFF = TBN * NBLK_MAIN
TAIL = OFF * 2
N_LINES = TBN * (NBLK_MAIN + N_TAIL_BLK)


def _fire_gathers(table_hbm, idx_buf, rows_buf, sem):
    return [
        pltpu.async_copy(
            table_hbm.at[idx_buf.at[pl.ds(i * GSPLIT, GSPLIT)]],
            rows_buf.at[pl.ds(i * GSPLIT, GSPLIT)],
            sem,
        )
        for i in range(N_GATHERS)
    ]


def _wait_gathers(table_hbm, idx_buf, rows_buf, sem):
    for i in range(N_GATHERS):
        pltpu.make_async_copy(
            table_hbm.at[idx_buf.at[pl.ds(i * GSPLIT, GSPLIT)]],
            rows_buf.at[pl.ds(i * GSPLIT, GSPLIT)],
            sem,
        ).wait()


def _accumulate(rows_buf, par_buf, pooled_v, c):
    for r in range(CHUNK):
        def jbody(j, accs):
            accs = list(accs)
            base = r * SEQ + j * UNROLL
            pvec = par_buf[pl.ds(base, 16)]
            for u in range(UNROLL):
                row = base + u
                off = (pvec[u] & 1) * HIDDEN
                for k in range(HREG):
                    accs[k] = accs[k] + rows_buf[row, pl.ds(off + k * 16, 16)]
            return tuple(accs)

        accs = lax.fori_loop(
            0, SEQ // UNROLL, jbody,
            tuple(jnp.zeros((16,), jnp.float32) for _ in range(HREG)),
        )
        out_base = (c * CHUNK + r) * HIDDEN
        for k in range(HREG):
            pooled_v[pl.ds(out_base + k * 16, 16)] = accs[k]


def _copy_par(par_buf, par_hold):
    for i in range(IDX_BUF // 16):
        par_hold[pl.ds(i * 16, 16)] = par_buf[pl.ds(i * 16, 16)]


def _sc_pool_kernel(xh_hbm, xp_hbm, table_hbm, out_hbm,
                    idx_a, idx_b, par_a, par_b, par_hold, rows_a, rows_b,
                    pooled_v, semg, semi):
    wid = lax.axis_index("s") * NC + lax.axis_index("c")
    base_flat = wid * (ROWS_PER_W * SEQ)

    def idx_src(c):
        return xh_hbm.at[pl.ds(base_flat + c * IDX_PER_CHUNK, IDX_PER_CHUNK)]

    def par_src(c):
        return xp_hbm.at[pl.ds(base_flat + c * IDX_PER_CHUNK, IDX_PER_CHUNK)]

    def clamp(c):
        return jnp.minimum(c, N_CHUNKS - 1)

    def fire_idx(c, idx_buf, par_buf):
        pltpu.async_copy(idx_src(c), idx_buf.at[pl.ds(0, IDX_PER_CHUNK)], semi)
        pltpu.async_copy(par_src(c), par_buf.at[pl.ds(0, IDX_PER_CHUNK)], semi)

    def wait_idx(c, idx_buf, par_buf):
        pltpu.make_async_copy(
            idx_src(c), idx_buf.at[pl.ds(0, IDX_PER_CHUNK)], semi).wait()
        pltpu.make_async_copy(
            par_src(c), par_buf.at[pl.ds(0, IDX_PER_CHUNK)], semi).wait()

    pltpu.sync_copy(idx_src(0), idx_a.at[pl.ds(0, IDX_PER_CHUNK)])
    pltpu.sync_copy(par_src(0), par_a.at[pl.ds(0, IDX_PER_CHUNK)])
    _fire_gathers(table_hbm, idx_a, rows_a, semg)
    fire_idx(1, idx_b, par_b)

    def body(i, carry):
        c0 = 2 * i
        wait_idx(clamp(c0 + 1), idx_b, par_b)
        _wait_gathers(table_hbm, idx_a, rows_a, semg)
        _fire_gathers(table_hbm, idx_b, rows_b, semg)
        _copy_par(par_a, par_hold)
        fire_idx(clamp(c0 + 2), idx_a, par_a)
        _accumulate(rows_a, par_hold, pooled_v, c0)
        wait_idx(clamp(c0 + 2), idx_a, par_a)
        _wait_gathers(table_hbm, idx_b, rows_b, semg)
        _fire_gathers(table_hbm, idx_a, rows_a, semg)
        _copy_par(par_b, par_hold)
        fire_idx(clamp(c0 + 3), idx_b, par_b)
        _accumulate(rows_b, par_hold, pooled_v, c0 + 1)
        return carry

    lax.fori_loop(0, N_CHUNKS // 2, body, 0)
    _wait_gathers(table_hbm, idx_a, rows_a, semg)
    wait_idx(N_CHUNKS - 1, idx_b, par_b)
    pltpu.sync_copy(
        pooled_v, out_hbm.at[pl.ds(wid * (ROWS_PER_W * HIDDEN), ROWS_PER_W * HIDDEN)]
    )


def _sc_pool(xh, xp, table):
    mesh = plsc.VectorSubcoreMesh(core_axis_name="c", subcore_axis_name="s")
    k = functools.partial(
        pl.kernel,
        mesh=mesh,
        out_type=jax.ShapeDtypeStruct((BATCH * HIDDEN,), jnp.float32),
        scratch_types=[
            pltpu.VMEM((IDX_BUF,), jnp.int32),
            pltpu.VMEM((IDX_BUF,), jnp.int32),
            pltpu.VMEM((IDX_BUF,), jnp.int32),
            pltpu.VMEM((IDX_BUF,), jnp.int32),
            pltpu.VMEM((IDX_BUF,), jnp.int32),
            pltpu.VMEM((IDX_PER_CHUNK, HPAD), jnp.float32),
            pltpu.VMEM((IDX_PER_CHUNK, HPAD), jnp.float32),
            pltpu.VMEM((ROWS_PER_W * HIDDEN,), jnp.float32),
            pltpu.SemaphoreType.DMA,
            pltpu.SemaphoreType.DMA,
        ],
    )(_sc_pool_kernel)
    return k(xh, xp, table)


def _tpad_kernel(tlo_ref, thi_ref, eye_ref, out_ref):
    dn = (((0,), (0,)), ((), ()))
    tlo = lax.dot_general(tlo_ref[...], eye_ref[...], dn,
                          preferred_element_type=jnp.float32)
    thi = lax.dot_general(thi_ref[...], eye_ref[...], dn,
                          preferred_element_type=jnp.float32)
    out_ref[...] = jnp.concatenate([tlo, thi], axis=1)


def _transpose_pair(table_t):
    eye = jnp.eye(HIDDEN, dtype=jnp.float32)
    return pl.pallas_call(
        _tpad_kernel,
        grid=(NBLK_MAIN + N_TAIL_BLK,),
        in_specs=[
            pl.BlockSpec(
                (HIDDEN, TBN),
                lambda i: (0, jnp.where(i < NBLK_MAIN, i, i + NBLK_MAIN)),
            ),
            pl.BlockSpec((HIDDEN, TBN), lambda i: (0, i + NBLK_MAIN)),
            pl.BlockSpec((HIDDEN, HIDDEN), lambda i: (0, 0)),
        ],
        out_specs=pl.BlockSpec((TBN, HPAD), lambda i: (i, 0)),
        out_shape=jax.ShapeDtypeStruct((N_LINES, HPAD), jnp.float32),
    )(table_t, table_t, eye)


def _fc_kernel(p_ref, wt_ref, b_ref, o_ref):
    p = p_ref[...] * (1.0 / SEQ)
    acc = jnp.dot(p, wt_ref[...], preferred_element_type=jnp.float32)
    o_ref[...] = jnp.maximum(acc + b_ref[...], 0.0)


def kernel(x, emb_table, W, b):
    xflat = x.astype(jnp.int32).reshape(-1)
    table_pairs = _transpose_pair(emb_table.T)
    is_tail = xflat >= TAIL
    xp = ((xflat >= OFF) & ~is_tail).astype(jnp.int32)
    xh = jnp.where(is_tail, xflat - TAIL + OFF, xflat - OFF * xp)
    pooled = _sc_pool(xh, xp, table_pairs).reshape(BATCH, HIDDEN)
    out = pl.pallas_call(
        _fc_kernel,
        out_shape=jax.ShapeDtypeStruct((BATCH, HIDDEN), jnp.float32),
    )(pooled, W.T, b.reshape(1, HIDDEN))
    return out
